# Initial kernel scaffold; baseline (speedup 1.0000x reference)
import jax
import jax.numpy as jnp
from jax.experimental import pallas as pl


def kernel(p, f, w1_0, g1_0, b1_0, w1_1, g1_1, b1_1, w2_0, g2_0, b2_0, w2_1, g2_1, b2_1, w2_2, g2_2, b2_2):
    raise NotImplementedError("write your pallas kernel here")


# speedup vs baseline: 12.0928x; 12.0928x over previous
import functools
import jax
import jax.numpy as jnp
from jax import lax
from jax.experimental import pallas as pl
from jax.experimental.pallas import tpu as pltpu
from jax.experimental.pallas import tpu_sc as plsc

_B, _N, _NP, _NS = 2, 8192, 2048, 32
_R2 = 0.15 * 0.15
_NW = 32
_CPW = (_B * _NP) // _NW
_WPB = _NP // _CPW
_NCHUNK = _N // 16


def _convs1_body(f_ref, w0_ref, g0_ref, b0_ref, w1_ref, g1_ref, b1_ref, o_ref):
    x = f_ref[0]
    h = jnp.dot(w0_ref[...], x, preferred_element_type=jnp.float32)
    h = jnp.maximum(h * g0_ref[...] + b0_ref[...], 0.0)
    y = jnp.dot(w1_ref[...], h, preferred_element_type=jnp.float32)
    o_ref[0] = jnp.maximum(y * g1_ref[...] + b1_ref[...], 0.0)


def _convs1(f, w0, g0, b0, w1, g1, b1):
    nt = 2048
    grid = (_B, _N // nt)
    full = lambda shape: pl.BlockSpec(shape, lambda b, j: (0, 0))
    return pl.pallas_call(
        _convs1_body,
        grid=grid,
        in_specs=[
            pl.BlockSpec((1, 32, nt), lambda b, j: (b, 0, j)),
            full((32, 32)), full((32, 1)), full((32, 1)),
            full((64, 32)), full((64, 1)), full((64, 1)),
        ],
        out_specs=pl.BlockSpec((1, 64, nt), lambda b, j: (b, 0, j)),
        out_shape=jax.ShapeDtypeStruct((_B, 64, _N), jnp.float32),
    )(f, w0, g0, b0, w1, g1, b1)


def _bf16r(x):
    b = plsc.bitcast(x, jnp.int32)
    r = (b + 0x7FFF + ((b >> 16) & 1)) & jnp.int32(-65536)
    return plsc.bitcast(r, jnp.float32)


def _bq_body(px_h, py_h, pz_h, sidx_h,
             npx_h, npy_h, npz_h, gidx_h, dpx_h, dpy_h, dpz_h,
             pxr, pyr, pzr, pnr, pxb, pyb, pzb, sidxv, cxv, cyv, czv, cnv,
             idxb, gidxv, dpxv, dpyv, dpzv):
    c = lax.axis_index("c")
    s = lax.axis_index("s")
    wid = s * 2 + c
    b = wid // _WPB
    pltpu.sync_copy(px_h.at[pl.ds(b * _N, _N)], pxr)
    pltpu.sync_copy(py_h.at[pl.ds(b * _N, _N)], pyr)
    pltpu.sync_copy(pz_h.at[pl.ds(b * _N, _N)], pzr)
    pltpu.sync_copy(sidx_h.at[pl.ds(wid * _CPW, _CPW)], sidxv)

    def norm_step(j, _):
        xv = pxr[pl.ds(j * 16, 16)]
        yv = pyr[pl.ds(j * 16, 16)]
        zv = pzr[pl.ds(j * 16, 16)]
        pnr[pl.ds(j * 16, 16)] = xv * xv + yv * yv + zv * zv
        pxb[pl.ds(j * 16, 16)] = _bf16r(xv)
        pyb[pl.ds(j * 16, 16)] = _bf16r(yv)
        pzb[pl.ds(j * 16, 16)] = _bf16r(zv)
        return 0
    lax.fori_loop(0, _NCHUNK, norm_step, 0)

    for g in range(_CPW // 16):
        iv = sidxv[pl.ds(g * 16, 16)]
        gx = plsc.load_gather(pxr, [iv])
        gy = plsc.load_gather(pyr, [iv])
        gz = plsc.load_gather(pzr, [iv])
        cxv[pl.ds(g * 16, 16)] = gx
        cyv[pl.ds(g * 16, 16)] = gy
        czv[pl.ds(g * 16, 16)] = gz
        cnv[pl.ds(g * 16, 16)] = gx * gx + gy * gy + gz * gz
    pltpu.sync_copy(cxv, npx_h.at[pl.ds(wid * _CPW, _CPW)])
    pltpu.sync_copy(cyv, npy_h.at[pl.ds(wid * _CPW, _CPW)])
    pltpu.sync_copy(czv, npz_h.at[pl.ds(wid * _CPW, _CPW)])

    lanes = lax.iota(jnp.int32, 16)
    zeros16 = jnp.zeros((16,), jnp.int32)

    def centroid(i, _):
        ib = jnp.full((16,), i, jnp.int32)
        cxb = plsc.load_gather(cxv, [ib])
        cyb = plsc.load_gather(cyv, [ib])
        czb = plsc.load_gather(czv, [ib])
        cnb = plsc.load_gather(cnv, [ib])
        cxq = _bf16r(cxb)
        cyq = _bf16r(cyb)
        czq = _bf16r(czb)
        idxb[pl.ds(0, 16)] = zeros16
        idxb[pl.ds(16, 16)] = zeros16

        def cond(st):
            cntv, ch = st
            return (jnp.max(cntv) < _NS) & (ch < _NCHUNK)

        def step(st):
            cntv, ch = st
            n0 = ch * 16
            xv = pxb[pl.ds(n0, 16)]
            yv = pyb[pl.ds(n0, 16)]
            zv = pzb[pl.ds(n0, 16)]
            nv = pnr[pl.ds(n0, 16)]
            d = (cnb + nv) - 2.0 * (cxq * xv + cyq * yv + czq * zv)
            m = d < _R2
            mi = m.astype(jnp.int32)
            pos = (cntv + jnp.cumsum(mi)) - 1
            sel = m & (pos < _NS)
            plsc.store_scatter(idxb, [pos], n0 + lanes, mask=sel)
            cntv = cntv + plsc.all_reduce_population_count(m)
            return (cntv, ch + 1)

        cntv, _ch = lax.while_loop(cond, step, (zeros16, jnp.int32(0)))

        iv00 = idxb[pl.ds(0, 16)]
        first = jnp.sum(jnp.where(lanes == 0, iv00, 0))
        base = i * _NS
        for g in range(2):
            iv0 = idxb[pl.ds(g * 16, 16)]
            iv = jnp.where((g * 16 + lanes) < cntv, iv0, first)
            gx = plsc.load_gather(pxr, [iv]) - cxb
            gy = plsc.load_gather(pyr, [iv]) - cyb
            gz = plsc.load_gather(pzr, [iv]) - czb
            dpxv[pl.ds(base + g * 16, 16)] = gx
            dpyv[pl.ds(base + g * 16, 16)] = gy
            dpzv[pl.ds(base + g * 16, 16)] = gz
            gidxv[pl.ds(base + g * 16, 16)] = iv + b * _N
        return 0

    lax.fori_loop(0, _CPW, centroid, 0)

    roff = wid * _CPW * _NS
    pltpu.sync_copy(gidxv, gidx_h.at[pl.ds(roff, _CPW * _NS)])
    pltpu.sync_copy(dpxv, dpx_h.at[pl.ds(roff, _CPW * _NS)])
    pltpu.sync_copy(dpyv, dpy_h.at[pl.ds(roff, _CPW * _NS)])
    pltpu.sync_copy(dpzv, dpz_h.at[pl.ds(roff, _CPW * _NS)])


def _ball_query(px, py, pz, sidx):
    m = _B * _NP
    r = m * _NS
    mesh = plsc.VectorSubcoreMesh(core_axis_name="c", subcore_axis_name="s")
    fn = pl.kernel(
        _bq_body,
        out_type=(
            jax.ShapeDtypeStruct((m,), jnp.float32),
            jax.ShapeDtypeStruct((m,), jnp.float32),
            jax.ShapeDtypeStruct((m,), jnp.float32),
            jax.ShapeDtypeStruct((r,), jnp.int32),
            jax.ShapeDtypeStruct((r,), jnp.float32),
            jax.ShapeDtypeStruct((r,), jnp.float32),
            jax.ShapeDtypeStruct((r,), jnp.float32),
        ),
        mesh=mesh,
        compiler_params=pltpu.CompilerParams(needs_layout_passes=False),
        scratch_types=[
            pltpu.VMEM((_N,), jnp.float32),
            pltpu.VMEM((_N,), jnp.float32),
            pltpu.VMEM((_N,), jnp.float32),
            pltpu.VMEM((_N,), jnp.float32),
            pltpu.VMEM((_N,), jnp.float32),
            pltpu.VMEM((_N,), jnp.float32),
            pltpu.VMEM((_N,), jnp.float32),
            pltpu.VMEM((_CPW,), jnp.int32),
            pltpu.VMEM((_CPW,), jnp.float32),
            pltpu.VMEM((_CPW,), jnp.float32),
            pltpu.VMEM((_CPW,), jnp.float32),
            pltpu.VMEM((_CPW,), jnp.float32),
            pltpu.VMEM((_NS,), jnp.int32),
            pltpu.VMEM((_CPW * _NS,), jnp.int32),
            pltpu.VMEM((_CPW * _NS,), jnp.float32),
            pltpu.VMEM((_CPW * _NS,), jnp.float32),
            pltpu.VMEM((_CPW * _NS,), jnp.float32),
        ],
    )
    return fn(px, py, pz, sidx)


_GROWS = 128


def _gather_body(gidx_h, f1_h, fj_h, idxv, rowsv, sem):
    c = lax.axis_index("c")
    s = lax.axis_index("s")
    wid = s * 2 + c
    base = wid * _CPW * _NS

    def rnd(r, _):
        off = base + r * _GROWS
        pltpu.sync_copy(gidx_h.at[pl.ds(off, _GROWS)], idxv)
        pltpu.async_copy(f1_h.at[idxv], rowsv, sem).wait()
        pltpu.sync_copy(rowsv, fj_h.at[pl.ds(off, _GROWS)])
        return 0

    lax.fori_loop(0, (_CPW * _NS) // _GROWS, rnd, 0)


def _gather_f1(gidx, f1_rows):
    r = _B * _NP * _NS
    mesh = plsc.VectorSubcoreMesh(core_axis_name="c", subcore_axis_name="s")
    fn = pl.kernel(
        _gather_body,
        out_type=jax.ShapeDtypeStruct((r, 64), jnp.float32),
        mesh=mesh,
        compiler_params=pltpu.CompilerParams(
            needs_layout_passes=False, use_tc_tiling_on_sc=False),
        scratch_types=[
            pltpu.VMEM((_GROWS,), jnp.int32),
            pltpu.VMEM((_GROWS, 64), jnp.float32),
            pltpu.SemaphoreType.DMA,
        ],
    )
    return fn(gidx, f1_rows)


def _fuse_body(dp_ref, fj_ref, w0_ref, g0_ref, b0_ref, w1_ref, g1_ref, b1_ref,
               w2_ref, g2_ref, b2_ref, o_ref):
    dp = dp_ref[...]
    e = jnp.dot(dp, w0_ref[...], preferred_element_type=jnp.float32)
    e = jnp.maximum(e * g0_ref[...] + b0_ref[...], 0.0)
    e = jnp.dot(e, w1_ref[...], preferred_element_type=jnp.float32)
    e = jnp.maximum(e * g1_ref[...] + b1_ref[...], 0.0)
    e = jnp.dot(e, w2_ref[...], preferred_element_type=jnp.float32)
    e = jnp.maximum(e * g2_ref[...] + b2_ref[...], 0.0)
    y = e + fj_ref[...]
    o_ref[...] = jnp.max(y.reshape(-1, _NS, 64), axis=1)


def _fuse(dp_rows, fj, w0, g0, b0, w1, g1, b1, w2, g2, b2):
    mt = 2048
    rows = _B * _NP * _NS
    grid = (rows // mt,)
    full = lambda shape: pl.BlockSpec(shape, lambda i: (0, 0))
    return pl.pallas_call(
        _fuse_body,
        grid=grid,
        in_specs=[
            pl.BlockSpec((mt, 3), lambda i: (i, 0)),
            pl.BlockSpec((mt, 64), lambda i: (i, 0)),
            full((3, 32)), full((1, 32)), full((1, 32)),
            full((32, 32)), full((1, 32)), full((1, 32)),
            full((32, 64)), full((1, 64)), full((1, 64)),
        ],
        out_specs=pl.BlockSpec((mt // _NS, 64), lambda i: (i, 0)),
        out_shape=jax.ShapeDtypeStruct((_B * _NP, 64), jnp.float32),
    )(dp_rows, fj, w0, g0, b0, w1, g1, b1, w2, g2, b2)


def kernel(p, f, w1_0, g1_0, b1_0, w1_1, g1_1, b1_1,
           w2_0, g2_0, b2_0, w2_1, g2_1, b2_1, w2_2, g2_2, b2_2):
    f1 = _convs1(f, w1_0, g1_0.reshape(32, 1), b1_0.reshape(32, 1),
                 w1_1, g1_1.reshape(64, 1), b1_1.reshape(64, 1))
    f1_rows = jnp.transpose(f1, (0, 2, 1)).reshape(_B * _N, 64)

    sidx = jax.random.randint(jax.random.key(42), (_B, _NP), 0, _N)
    sidx = sidx.astype(jnp.int32).reshape(-1)

    px = p[:, :, 0].reshape(-1)
    py = p[:, :, 1].reshape(-1)
    pz = p[:, :, 2].reshape(-1)
    npx, npy, npz, gidx, dpx, dpy, dpz = _ball_query(px, py, pz, sidx)

    fj = _gather_f1(gidx, f1_rows)
    dp_rows = jnp.stack([dpx, dpy, dpz], axis=-1)

    out_rows = _fuse(dp_rows, fj,
                     w2_0.T, g2_0.reshape(1, 32), b2_0.reshape(1, 32),
                     w2_1.T, g2_1.reshape(1, 32), b2_1.reshape(1, 32),
                     w2_2.T, g2_2.reshape(1, 64), b2_2.reshape(1, 64))

    new_p = jnp.stack([npx, npy, npz], axis=-1).reshape(_B, _NP, 3)
    out = out_rows.reshape(_B, _NP, 64).transpose(0, 2, 1)
    return new_p, out


# speedup vs baseline: 16.4858x; 1.3633x over previous
import functools
import jax
import jax.numpy as jnp
from jax import lax
from jax.experimental import pallas as pl
from jax.experimental.pallas import tpu as pltpu
from jax.experimental.pallas import tpu_sc as plsc

_B, _N, _NP, _NS = 2, 8192, 2048, 32
_R2 = 0.15 * 0.15
_NW = 32
_CPW = (_B * _NP) // _NW
_WPB = _NP // _CPW
_NCHUNK = _N // 16


def _convs1_body(f_ref, w0_ref, g0_ref, b0_ref, w1_ref, g1_ref, b1_ref, o_ref):
    x = f_ref[0]
    h = jnp.dot(w0_ref[...], x, preferred_element_type=jnp.float32)
    h = jnp.maximum(h * g0_ref[...] + b0_ref[...], 0.0)
    y = jnp.dot(w1_ref[...], h, preferred_element_type=jnp.float32)
    o_ref[0] = jnp.maximum(y * g1_ref[...] + b1_ref[...], 0.0)


def _convs1(f, w0, g0, b0, w1, g1, b1):
    nt = 2048
    grid = (_B, _N // nt)
    full = lambda shape: pl.BlockSpec(shape, lambda b, j: (0, 0))
    return pl.pallas_call(
        _convs1_body,
        grid=grid,
        in_specs=[
            pl.BlockSpec((1, 32, nt), lambda b, j: (b, 0, j)),
            full((32, 32)), full((32, 1)), full((32, 1)),
            full((64, 32)), full((64, 1)), full((64, 1)),
        ],
        out_specs=pl.BlockSpec((1, 64, nt), lambda b, j: (b, 0, j)),
        out_shape=jax.ShapeDtypeStruct((_B, 64, _N), jnp.float32),
    )(f, w0, g0, b0, w1, g1, b1)


def _bf16r(x):
    b = plsc.bitcast(x, jnp.int32)
    r = (b + 0x7FFF + ((b >> 16) & 1)) & jnp.int32(-65536)
    return plsc.bitcast(r, jnp.float32)


def _bq_body(px_h, py_h, pz_h, sidx_h,
             npx_h, npy_h, npz_h, gidx_h, dpx_h, dpy_h, dpz_h,
             pxr, pyr, pzr, pnr, pxb, pyb, pzb, sidxv, cxv, cyv, czv, cnv,
             idxb, gidxv, dpxv, dpyv, dpzv):
    c = lax.axis_index("c")
    s = lax.axis_index("s")
    wid = s * 2 + c
    b = wid // _WPB
    pltpu.sync_copy(px_h.at[pl.ds(b * _N, _N)], pxr)
    pltpu.sync_copy(py_h.at[pl.ds(b * _N, _N)], pyr)
    pltpu.sync_copy(pz_h.at[pl.ds(b * _N, _N)], pzr)
    pltpu.sync_copy(sidx_h.at[pl.ds(wid * _CPW, _CPW)], sidxv)

    def norm_step(j, _):
        xv = pxr[pl.ds(j * 16, 16)]
        yv = pyr[pl.ds(j * 16, 16)]
        zv = pzr[pl.ds(j * 16, 16)]
        pnr[pl.ds(j * 16, 16)] = xv * xv + yv * yv + zv * zv
        pxb[pl.ds(j * 16, 16)] = _bf16r(xv)
        pyb[pl.ds(j * 16, 16)] = _bf16r(yv)
        pzb[pl.ds(j * 16, 16)] = _bf16r(zv)
        return 0
    lax.fori_loop(0, _NCHUNK, norm_step, 0)

    for g in range(_CPW // 16):
        iv = sidxv[pl.ds(g * 16, 16)]
        gx = plsc.load_gather(pxr, [iv])
        gy = plsc.load_gather(pyr, [iv])
        gz = plsc.load_gather(pzr, [iv])
        cxv[pl.ds(g * 16, 16)] = gx
        cyv[pl.ds(g * 16, 16)] = gy
        czv[pl.ds(g * 16, 16)] = gz
        cnv[pl.ds(g * 16, 16)] = gx * gx + gy * gy + gz * gz
    pltpu.sync_copy(cxv, npx_h.at[pl.ds(wid * _CPW, _CPW)])
    pltpu.sync_copy(cyv, npy_h.at[pl.ds(wid * _CPW, _CPW)])
    pltpu.sync_copy(czv, npz_h.at[pl.ds(wid * _CPW, _CPW)])

    lanes = lax.iota(jnp.int32, 16)
    zeros16 = jnp.zeros((16,), jnp.int32)

    def centroid(i, _):
        ib = jnp.full((16,), i, jnp.int32)
        cxb = plsc.load_gather(cxv, [ib])
        cyb = plsc.load_gather(cyv, [ib])
        czb = plsc.load_gather(czv, [ib])
        cnb = plsc.load_gather(cnv, [ib])
        cxq = _bf16r(cxb)
        cyq = _bf16r(cyb)
        czq = _bf16r(czb)
        idxb[pl.ds(0, 16)] = zeros16
        idxb[pl.ds(16, 16)] = zeros16

        def cond(st):
            cntv, grp = st
            return (jnp.max(cntv) < _NS) & (grp < _NCHUNK // 8)

        def step(st):
            cntv, grp = st
            nb = grp * 128
            for k in range(8):
                n0 = nb + k * 16
                xv = pxb[pl.ds(n0, 16)]
                yv = pyb[pl.ds(n0, 16)]
                zv = pzb[pl.ds(n0, 16)]
                nv = pnr[pl.ds(n0, 16)]
                d = (cnb + nv) - 2.0 * (cxq * xv + cyq * yv + czq * zv)
                m = d < _R2
                mi = m.astype(jnp.int32)
                pos = (cntv + jnp.cumsum(mi)) - 1
                sel = m & (pos < _NS)
                plsc.store_scatter(idxb, [pos], n0 + lanes, mask=sel)
                cntv = cntv + plsc.all_reduce_population_count(m)
            return (cntv, grp + 1)

        cntv, _ch = lax.while_loop(cond, step, (zeros16, jnp.int32(0)))

        iv00 = idxb[pl.ds(0, 16)]
        first = jnp.sum(jnp.where(lanes == 0, iv00, 0))
        base = i * _NS
        for g in range(2):
            iv0 = idxb[pl.ds(g * 16, 16)]
            iv = jnp.where((g * 16 + lanes) < cntv, iv0, first)
            gx = plsc.load_gather(pxr, [iv]) - cxb
            gy = plsc.load_gather(pyr, [iv]) - cyb
            gz = plsc.load_gather(pzr, [iv]) - czb
            dpxv[pl.ds(base + g * 16, 16)] = gx
            dpyv[pl.ds(base + g * 16, 16)] = gy
            dpzv[pl.ds(base + g * 16, 16)] = gz
            gidxv[pl.ds(base + g * 16, 16)] = iv + b * _N
        return 0

    lax.fori_loop(0, _CPW, centroid, 0)

    roff = wid * _CPW * _NS
    pltpu.sync_copy(gidxv, gidx_h.at[pl.ds(roff, _CPW * _NS)])
    pltpu.sync_copy(dpxv, dpx_h.at[pl.ds(roff, _CPW * _NS)])
    pltpu.sync_copy(dpyv, dpy_h.at[pl.ds(roff, _CPW * _NS)])
    pltpu.sync_copy(dpzv, dpz_h.at[pl.ds(roff, _CPW * _NS)])


def _ball_query(px, py, pz, sidx):
    m = _B * _NP
    r = m * _NS
    mesh = plsc.VectorSubcoreMesh(core_axis_name="c", subcore_axis_name="s")
    fn = pl.kernel(
        _bq_body,
        out_type=(
            jax.ShapeDtypeStruct((m,), jnp.float32),
            jax.ShapeDtypeStruct((m,), jnp.float32),
            jax.ShapeDtypeStruct((m,), jnp.float32),
            jax.ShapeDtypeStruct((r,), jnp.int32),
            jax.ShapeDtypeStruct((r,), jnp.float32),
            jax.ShapeDtypeStruct((r,), jnp.float32),
            jax.ShapeDtypeStruct((r,), jnp.float32),
        ),
        mesh=mesh,
        compiler_params=pltpu.CompilerParams(needs_layout_passes=False),
        scratch_types=[
            pltpu.VMEM((_N,), jnp.float32),
            pltpu.VMEM((_N,), jnp.float32),
            pltpu.VMEM((_N,), jnp.float32),
            pltpu.VMEM((_N,), jnp.float32),
            pltpu.VMEM((_N,), jnp.float32),
            pltpu.VMEM((_N,), jnp.float32),
            pltpu.VMEM((_N,), jnp.float32),
            pltpu.VMEM((_CPW,), jnp.int32),
            pltpu.VMEM((_CPW,), jnp.float32),
            pltpu.VMEM((_CPW,), jnp.float32),
            pltpu.VMEM((_CPW,), jnp.float32),
            pltpu.VMEM((_CPW,), jnp.float32),
            pltpu.VMEM((_NS,), jnp.int32),
            pltpu.VMEM((_CPW * _NS,), jnp.int32),
            pltpu.VMEM((_CPW * _NS,), jnp.float32),
            pltpu.VMEM((_CPW * _NS,), jnp.float32),
            pltpu.VMEM((_CPW * _NS,), jnp.float32),
        ],
    )
    return fn(px, py, pz, sidx)


_GROWS = 128


def _gather_body(gidx_h, f1_h, fj_h, idxv, rowsv, sem):
    c = lax.axis_index("c")
    s = lax.axis_index("s")
    wid = s * 2 + c
    base = wid * _CPW * _NS

    def rnd(r, _):
        off = base + r * _GROWS
        pltpu.sync_copy(gidx_h.at[pl.ds(off, _GROWS)], idxv)
        pltpu.async_copy(f1_h.at[idxv], rowsv, sem).wait()
        pltpu.sync_copy(rowsv, fj_h.at[pl.ds(off, _GROWS)])
        return 0

    lax.fori_loop(0, (_CPW * _NS) // _GROWS, rnd, 0)


def _gather_f1(gidx, f1_rows):
    r = _B * _NP * _NS
    mesh = plsc.VectorSubcoreMesh(core_axis_name="c", subcore_axis_name="s")
    fn = pl.kernel(
        _gather_body,
        out_type=jax.ShapeDtypeStruct((r, 64), jnp.float32),
        mesh=mesh,
        compiler_params=pltpu.CompilerParams(
            needs_layout_passes=False, use_tc_tiling_on_sc=False),
        scratch_types=[
            pltpu.VMEM((_GROWS,), jnp.int32),
            pltpu.VMEM((_GROWS, 64), jnp.float32),
            pltpu.SemaphoreType.DMA,
        ],
    )
    return fn(gidx, f1_rows)


def _fuse_body(dp_ref, fj_ref, w0_ref, g0_ref, b0_ref, w1_ref, g1_ref, b1_ref,
               w2_ref, g2_ref, b2_ref, o_ref):
    dp = dp_ref[...]
    e = jnp.dot(dp, w0_ref[...], preferred_element_type=jnp.float32)
    e = jnp.maximum(e * g0_ref[...] + b0_ref[...], 0.0)
    e = jnp.dot(e, w1_ref[...], preferred_element_type=jnp.float32)
    e = jnp.maximum(e * g1_ref[...] + b1_ref[...], 0.0)
    e = jnp.dot(e, w2_ref[...], preferred_element_type=jnp.float32)
    e = jnp.maximum(e * g2_ref[...] + b2_ref[...], 0.0)
    y = e + fj_ref[...]
    o_ref[...] = jnp.max(y.reshape(-1, _NS, 64), axis=1)


def _fuse(dp_rows, fj, w0, g0, b0, w1, g1, b1, w2, g2, b2):
    mt = 2048
    rows = _B * _NP * _NS
    grid = (rows // mt,)
    full = lambda shape: pl.BlockSpec(shape, lambda i: (0, 0))
    return pl.pallas_call(
        _fuse_body,
        grid=grid,
        in_specs=[
            pl.BlockSpec((mt, 3), lambda i: (i, 0)),
            pl.BlockSpec((mt, 64), lambda i: (i, 0)),
            full((3, 32)), full((1, 32)), full((1, 32)),
            full((32, 32)), full((1, 32)), full((1, 32)),
            full((32, 64)), full((1, 64)), full((1, 64)),
        ],
        out_specs=pl.BlockSpec((mt // _NS, 64), lambda i: (i, 0)),
        out_shape=jax.ShapeDtypeStruct((_B * _NP, 64), jnp.float32),
    )(dp_rows, fj, w0, g0, b0, w1, g1, b1, w2, g2, b2)


def kernel(p, f, w1_0, g1_0, b1_0, w1_1, g1_1, b1_1,
           w2_0, g2_0, b2_0, w2_1, g2_1, b2_1, w2_2, g2_2, b2_2):
    f1 = _convs1(f, w1_0, g1_0.reshape(32, 1), b1_0.reshape(32, 1),
                 w1_1, g1_1.reshape(64, 1), b1_1.reshape(64, 1))
    f1_rows = jnp.transpose(f1, (0, 2, 1)).reshape(_B * _N, 64)

    sidx = jax.random.randint(jax.random.key(42), (_B, _NP), 0, _N)
    sidx = sidx.astype(jnp.int32).reshape(-1)

    px = p[:, :, 0].reshape(-1)
    py = p[:, :, 1].reshape(-1)
    pz = p[:, :, 2].reshape(-1)
    npx, npy, npz, gidx, dpx, dpy, dpz = _ball_query(px, py, pz, sidx)

    fj = _gather_f1(gidx, f1_rows)
    dp_rows = jnp.stack([dpx, dpy, dpz], axis=-1)

    out_rows = _fuse(dp_rows, fj,
                     w2_0.T, g2_0.reshape(1, 32), b2_0.reshape(1, 32),
                     w2_1.T, g2_1.reshape(1, 32), b2_1.reshape(1, 32),
                     w2_2.T, g2_2.reshape(1, 64), b2_2.reshape(1, 64))

    new_p = jnp.stack([npx, npy, npz], axis=-1).reshape(_B, _NP, 3)
    out = out_rows.reshape(_B, _NP, 64).transpose(0, 2, 1)
    return new_p, out


# speedup vs baseline: 17.0901x; 1.0367x over previous
import functools
import jax
import jax.numpy as jnp
from jax import lax
from jax.experimental import pallas as pl
from jax.experimental.pallas import tpu as pltpu
from jax.experimental.pallas import tpu_sc as plsc

_B, _N, _NP, _NS = 2, 8192, 2048, 32
_R2 = 0.15 * 0.15
_NW = 32
_CPW = (_B * _NP) // _NW
_WPB = _NP // _CPW
_NCHUNK = _N // 16


def _convs1_body(f_ref, w0_ref, g0_ref, b0_ref, w1_ref, g1_ref, b1_ref, o_ref):
    x = f_ref[0]
    h = jnp.dot(w0_ref[...], x, preferred_element_type=jnp.float32)
    h = jnp.maximum(h * g0_ref[...] + b0_ref[...], 0.0)
    y = jnp.dot(w1_ref[...], h, preferred_element_type=jnp.float32)
    o_ref[0] = jnp.maximum(y * g1_ref[...] + b1_ref[...], 0.0)


def _convs1(f, w0, g0, b0, w1, g1, b1):
    nt = 2048
    grid = (_B, _N // nt)
    full = lambda shape: pl.BlockSpec(shape, lambda b, j: (0, 0))
    return pl.pallas_call(
        _convs1_body,
        grid=grid,
        in_specs=[
            pl.BlockSpec((1, 32, nt), lambda b, j: (b, 0, j)),
            full((32, 32)), full((32, 1)), full((32, 1)),
            full((64, 32)), full((64, 1)), full((64, 1)),
        ],
        out_specs=pl.BlockSpec((1, 64, nt), lambda b, j: (b, 0, j)),
        out_shape=jax.ShapeDtypeStruct((_B, 64, _N), jnp.float32),
    )(f, w0, g0, b0, w1, g1, b1)


def _bf16r(x):
    b = plsc.bitcast(x, jnp.int32)
    r = (b + 0x7FFF + ((b >> 16) & 1)) & jnp.int32(-65536)
    return plsc.bitcast(r, jnp.float32)


def _bq_body(px_h, py_h, pz_h, sidx_h,
             npx_h, npy_h, npz_h, gidx_h, dpx_h, dpy_h, dpz_h,
             pxr, pyr, pzr, pnr, pxb, pzb, sidxv, cxv, cyv, czv, cnv,
             idxb, gidxv, dpxv, dpyv, dpzv):
    c = lax.axis_index("c")
    s = lax.axis_index("s")
    wid = s * 2 + c
    b = wid // _WPB
    pltpu.sync_copy(px_h.at[pl.ds(b * _N, _N)], pxr)
    pltpu.sync_copy(py_h.at[pl.ds(b * _N, _N)], pyr)
    pltpu.sync_copy(pz_h.at[pl.ds(b * _N, _N)], pzr)
    pltpu.sync_copy(sidx_h.at[pl.ds(wid * _CPW, _CPW)], sidxv)

    def norm_step(j, _):
        xv = pxr[pl.ds(j * 16, 16)]
        yv = pyr[pl.ds(j * 16, 16)]
        zv = pzr[pl.ds(j * 16, 16)]
        pnr[pl.ds(j * 16, 16)] = xv * xv + yv * yv + zv * zv
        xb = plsc.bitcast(_bf16r(xv), jnp.int32)
        yb = plsc.bitcast(_bf16r(yv), jnp.int32)
        pxb[pl.ds(j * 16, 16)] = (xb & jnp.int32(-65536)) | ((yb >> 16) & 0xFFFF)
        pzb[pl.ds(j * 16, 16)] = _bf16r(zv)
        return 0
    lax.fori_loop(0, _NCHUNK, norm_step, 0)

    for g in range(_CPW // 16):
        iv = sidxv[pl.ds(g * 16, 16)]
        gx = plsc.load_gather(pxr, [iv])
        gy = plsc.load_gather(pyr, [iv])
        gz = plsc.load_gather(pzr, [iv])
        cxv[pl.ds(g * 16, 16)] = gx
        cyv[pl.ds(g * 16, 16)] = gy
        czv[pl.ds(g * 16, 16)] = gz
        cnv[pl.ds(g * 16, 16)] = gx * gx + gy * gy + gz * gz
    pltpu.sync_copy(cxv, npx_h.at[pl.ds(wid * _CPW, _CPW)])
    pltpu.sync_copy(cyv, npy_h.at[pl.ds(wid * _CPW, _CPW)])
    pltpu.sync_copy(czv, npz_h.at[pl.ds(wid * _CPW, _CPW)])

    lanes = lax.iota(jnp.int32, 16)
    zeros16 = jnp.zeros((16,), jnp.int32)

    def centroid(i, _):
        ib = jnp.full((16,), i, jnp.int32)
        cxb = plsc.load_gather(cxv, [ib])
        cyb = plsc.load_gather(cyv, [ib])
        czb = plsc.load_gather(czv, [ib])
        cnb = plsc.load_gather(cnv, [ib])
        cxq = _bf16r(cxb) * 2.0
        cyq = _bf16r(cyb) * 2.0
        czq = _bf16r(czb) * 2.0
        idxb[pl.ds(0, 16)] = zeros16
        idxb[pl.ds(16, 16)] = zeros16

        def cond(st):
            cntv, grp = st
            return (jnp.max(cntv) < _NS) & (grp < _NCHUNK // 8)

        def step(st):
            cntv, grp = st
            nb = grp * 128
            for k in range(8):
                n0 = nb + k * 16
                xy = pxb[pl.ds(n0, 16)]
                xv = plsc.bitcast(xy & jnp.int32(-65536), jnp.float32)
                yv = plsc.bitcast(xy << 16, jnp.float32)
                zv = pzb[pl.ds(n0, 16)]
                nv = pnr[pl.ds(n0, 16)]
                d = (cnb + nv) - (cxq * xv + cyq * yv + czq * zv)
                m = d < _R2
                mi = m.astype(jnp.int32)
                pos = (cntv + jnp.cumsum(mi)) - 1
                sel = m & (pos < _NS)
                plsc.store_scatter(idxb, [pos], n0 + lanes, mask=sel)
                cntv = cntv + plsc.all_reduce_population_count(m)
            return (cntv, grp + 1)

        cntv, _ch = lax.while_loop(cond, step, (zeros16, jnp.int32(0)))

        iv00 = idxb[pl.ds(0, 16)]
        first = jnp.sum(jnp.where(lanes == 0, iv00, 0))
        base = i * _NS
        for g in range(2):
            iv0 = idxb[pl.ds(g * 16, 16)]
            iv = jnp.where((g * 16 + lanes) < cntv, iv0, first)
            gx = plsc.load_gather(pxr, [iv]) - cxb
            gy = plsc.load_gather(pyr, [iv]) - cyb
            gz = plsc.load_gather(pzr, [iv]) - czb
            dpxv[pl.ds(base + g * 16, 16)] = gx
            dpyv[pl.ds(base + g * 16, 16)] = gy
            dpzv[pl.ds(base + g * 16, 16)] = gz
            gidxv[pl.ds(base + g * 16, 16)] = iv + b * _N
        return 0

    lax.fori_loop(0, _CPW, centroid, 0)

    roff = wid * _CPW * _NS
    pltpu.sync_copy(gidxv, gidx_h.at[pl.ds(roff, _CPW * _NS)])
    pltpu.sync_copy(dpxv, dpx_h.at[pl.ds(roff, _CPW * _NS)])
    pltpu.sync_copy(dpyv, dpy_h.at[pl.ds(roff, _CPW * _NS)])
    pltpu.sync_copy(dpzv, dpz_h.at[pl.ds(roff, _CPW * _NS)])


def _ball_query(px, py, pz, sidx):
    m = _B * _NP
    r = m * _NS
    mesh = plsc.VectorSubcoreMesh(core_axis_name="c", subcore_axis_name="s")
    fn = pl.kernel(
        _bq_body,
        out_type=(
            jax.ShapeDtypeStruct((m,), jnp.float32),
            jax.ShapeDtypeStruct((m,), jnp.float32),
            jax.ShapeDtypeStruct((m,), jnp.float32),
            jax.ShapeDtypeStruct((r,), jnp.int32),
            jax.ShapeDtypeStruct((r,), jnp.float32),
            jax.ShapeDtypeStruct((r,), jnp.float32),
            jax.ShapeDtypeStruct((r,), jnp.float32),
        ),
        mesh=mesh,
        compiler_params=pltpu.CompilerParams(needs_layout_passes=False),
        scratch_types=[
            pltpu.VMEM((_N,), jnp.float32),
            pltpu.VMEM((_N,), jnp.float32),
            pltpu.VMEM((_N,), jnp.float32),
            pltpu.VMEM((_N,), jnp.float32),
            pltpu.VMEM((_N,), jnp.int32),
            pltpu.VMEM((_N,), jnp.float32),
            pltpu.VMEM((_CPW,), jnp.int32),
            pltpu.VMEM((_CPW,), jnp.float32),
            pltpu.VMEM((_CPW,), jnp.float32),
            pltpu.VMEM((_CPW,), jnp.float32),
            pltpu.VMEM((_CPW,), jnp.float32),
            pltpu.VMEM((_NS,), jnp.int32),
            pltpu.VMEM((_CPW * _NS,), jnp.int32),
            pltpu.VMEM((_CPW * _NS,), jnp.float32),
            pltpu.VMEM((_CPW * _NS,), jnp.float32),
            pltpu.VMEM((_CPW * _NS,), jnp.float32),
        ],
    )
    return fn(px, py, pz, sidx)


_GROWS = 128


def _gather_body(gidx_h, f1_h, fj_h, idxv, rowsv, sem):
    c = lax.axis_index("c")
    s = lax.axis_index("s")
    wid = s * 2 + c
    base = wid * _CPW * _NS

    def rnd(r, _):
        off = base + r * _GROWS
        pltpu.sync_copy(gidx_h.at[pl.ds(off, _GROWS)], idxv)
        pltpu.async_copy(f1_h.at[idxv], rowsv, sem).wait()
        pltpu.sync_copy(rowsv, fj_h.at[pl.ds(off, _GROWS)])
        return 0

    lax.fori_loop(0, (_CPW * _NS) // _GROWS, rnd, 0)


def _gather_f1(gidx, f1_rows):
    r = _B * _NP * _NS
    mesh = plsc.VectorSubcoreMesh(core_axis_name="c", subcore_axis_name="s")
    fn = pl.kernel(
        _gather_body,
        out_type=jax.ShapeDtypeStruct((r, 64), jnp.float32),
        mesh=mesh,
        compiler_params=pltpu.CompilerParams(
            needs_layout_passes=False, use_tc_tiling_on_sc=False),
        scratch_types=[
            pltpu.VMEM((_GROWS,), jnp.int32),
            pltpu.VMEM((_GROWS, 64), jnp.float32),
            pltpu.SemaphoreType.DMA,
        ],
    )
    return fn(gidx, f1_rows)


def _fuse_body(dp_ref, fj_ref, w0_ref, g0_ref, b0_ref, w1_ref, g1_ref, b1_ref,
               w2_ref, g2_ref, b2_ref, o_ref):
    dp = dp_ref[...]
    e = jnp.dot(dp, w0_ref[...], preferred_element_type=jnp.float32)
    e = jnp.maximum(e * g0_ref[...] + b0_ref[...], 0.0)
    e = jnp.dot(e, w1_ref[...], preferred_element_type=jnp.float32)
    e = jnp.maximum(e * g1_ref[...] + b1_ref[...], 0.0)
    e = jnp.dot(e, w2_ref[...], preferred_element_type=jnp.float32)
    e = jnp.maximum(e * g2_ref[...] + b2_ref[...], 0.0)
    y = e + fj_ref[...]
    o_ref[...] = jnp.max(y.reshape(-1, _NS, 64), axis=1)


def _fuse(dp_rows, fj, w0, g0, b0, w1, g1, b1, w2, g2, b2):
    mt = 2048
    rows = _B * _NP * _NS
    grid = (rows // mt,)
    full = lambda shape: pl.BlockSpec(shape, lambda i: (0, 0))
    return pl.pallas_call(
        _fuse_body,
        grid=grid,
        in_specs=[
            pl.BlockSpec((mt, 3), lambda i: (i, 0)),
            pl.BlockSpec((mt, 64), lambda i: (i, 0)),
            full((3, 32)), full((1, 32)), full((1, 32)),
            full((32, 32)), full((1, 32)), full((1, 32)),
            full((32, 64)), full((1, 64)), full((1, 64)),
        ],
        out_specs=pl.BlockSpec((mt // _NS, 64), lambda i: (i, 0)),
        out_shape=jax.ShapeDtypeStruct((_B * _NP, 64), jnp.float32),
    )(dp_rows, fj, w0, g0, b0, w1, g1, b1, w2, g2, b2)


def kernel(p, f, w1_0, g1_0, b1_0, w1_1, g1_1, b1_1,
           w2_0, g2_0, b2_0, w2_1, g2_1, b2_1, w2_2, g2_2, b2_2):
    f1 = _convs1(f, w1_0, g1_0.reshape(32, 1), b1_0.reshape(32, 1),
                 w1_1, g1_1.reshape(64, 1), b1_1.reshape(64, 1))
    f1_rows = jnp.transpose(f1, (0, 2, 1)).reshape(_B * _N, 64)

    sidx = jax.random.randint(jax.random.key(42), (_B, _NP), 0, _N)
    sidx = sidx.astype(jnp.int32).reshape(-1)

    px = p[:, :, 0].reshape(-1)
    py = p[:, :, 1].reshape(-1)
    pz = p[:, :, 2].reshape(-1)
    npx, npy, npz, gidx, dpx, dpy, dpz = _ball_query(px, py, pz, sidx)

    fj = _gather_f1(gidx, f1_rows)
    dp_rows = jnp.stack([dpx, dpy, dpz], axis=-1)

    out_rows = _fuse(dp_rows, fj,
                     w2_0.T, g2_0.reshape(1, 32), b2_0.reshape(1, 32),
                     w2_1.T, g2_1.reshape(1, 32), b2_1.reshape(1, 32),
                     w2_2.T, g2_2.reshape(1, 64), b2_2.reshape(1, 64))

    new_p = jnp.stack([npx, npy, npz], axis=-1).reshape(_B, _NP, 3)
    out = out_rows.reshape(_B, _NP, 64).transpose(0, 2, 1)
    return new_p, out
